# baseline (speedup 1.0000x reference)
import jax
import jax.numpy as jnp
from jax import lax
from jax.experimental import pallas as pl
from jax.experimental.pallas import tpu as pltpu
from jax.experimental.pallas import tpu_sc as plsc

NB = 4096
T = 50
B = NB * T
D = 128
NC, NS = 2, 16
NW = NC * NS
BPW = B // NW
CHUNK = 400
NBUF = 2
NCHUNK = BPW // CHUNK
NGROUP = NCHUNK // NBUF


def _gather_body(table_hbm, idx_hbm, out_hbm, table_sp, idx_v, rows0, rows1,
                 gsem, ssem0, ssem1):
    rows = (rows0, rows1)
    ssem = (ssem0, ssem1)
    sid = lax.axis_index("s")
    wid = sid * NC + lax.axis_index("c")
    base = wid * BPW

    rows_per_tile = 1024 // NS
    pltpu.sync_copy(table_hbm.at[pl.ds(sid * rows_per_tile, rows_per_tile)],
                    table_sp.at[pl.ds(sid * rows_per_tile, rows_per_tile)])
    pltpu.sync_copy(idx_hbm.at[pl.ds(base, BPW)], idx_v)
    plsc.subcore_barrier()

    def gather(c, b):
        return pltpu.make_async_copy(
            table_sp.at[idx_v.at[pl.ds(c * CHUNK, CHUNK)]], rows[b], gsem)

    def scatter(c, b):
        return pltpu.make_async_copy(
            rows[b], out_hbm.at[pl.ds(base + c * CHUNK, CHUNK)], ssem[b])

    gather(0, 0).start()

    @pl.loop(0, NGROUP)
    def _grp(g):
        c0 = g * NBUF
        gather(c0, 0).wait()
        scatter(c0, 0).start()

        @pl.when(g > 0)
        def _():
            scatter(c0 - 1, 1).wait()

        gather(c0 + 1, 1).start()
        gather(c0 + 1, 1).wait()
        scatter(c0 + 1, 1).start()
        scatter(c0, 0).wait()

        @pl.when(g < NGROUP - 1)
        def _():
            gather(c0 + 2, 0).start()

    scatter(NCHUNK - 1, 1).wait()


@jax.jit
def _sc_gather(tmp_enc, tidx):
    mesh = plsc.VectorSubcoreMesh(core_axis_name="c", subcore_axis_name="s")
    return pl.kernel(
        _gather_body,
        out_type=jax.ShapeDtypeStruct((B, D), jnp.float32),
        mesh=mesh,
        compiler_params=pltpu.CompilerParams(
            disable_bounds_checks=True, disable_semaphore_checks=True),
        scratch_types=[
            pltpu.VMEM_SHARED((1024, D), jnp.float32),
            pltpu.VMEM((BPW,), jnp.int32),
            pltpu.VMEM((CHUNK, D), jnp.float32),
            pltpu.VMEM((CHUNK, D), jnp.float32),
            pltpu.SemaphoreType.DMA,
            pltpu.SemaphoreType.DMA,
            pltpu.SemaphoreType.DMA,
        ],
    )(tmp_enc, tidx)


def kernel(time, tmp_enc):
    tidx = time.T.reshape(-1).astype(jnp.int32)
    out = _sc_gather(tmp_enc, tidx)
    return out.reshape(T, NB, D).transpose(1, 0, 2)


# speedup vs baseline: 1.0030x; 1.0030x over previous
import jax
import jax.numpy as jnp
from jax import lax
from jax.experimental import pallas as pl
from jax.experimental.pallas import tpu as pltpu
from jax.experimental.pallas import tpu_sc as plsc

NB = 4096
T = 50
B = NB * T
D = 128
NC, NS = 2, 16
NW = NC * NS
BPW = B // NW
CHUNK = 320
NBUF = 2
NCHUNK = BPW // CHUNK
NGROUP = NCHUNK // NBUF


def _gather_body(table_hbm, idx_hbm, out_hbm, table_sp, idx_v, rows0, rows1,
                 gsem, ssem0, ssem1):
    rows = (rows0, rows1)
    ssem = (ssem0, ssem1)
    sid = lax.axis_index("s")
    wid = sid * NC + lax.axis_index("c")
    base = wid * BPW

    rows_per_tile = 1024 // NS
    pltpu.sync_copy(table_hbm.at[pl.ds(sid * rows_per_tile, rows_per_tile)],
                    table_sp.at[pl.ds(sid * rows_per_tile, rows_per_tile)])
    pltpu.sync_copy(idx_hbm.at[pl.ds(base, BPW)], idx_v)
    plsc.subcore_barrier()

    def gather(c, b):
        return pltpu.make_async_copy(
            table_sp.at[idx_v.at[pl.ds(c * CHUNK, CHUNK)]], rows[b], gsem)

    def scatter(c, b):
        return pltpu.make_async_copy(
            rows[b], out_hbm.at[pl.ds(base + c * CHUNK, CHUNK)], ssem[b])

    gather(0, 0).start()

    @pl.loop(0, NGROUP)
    def _grp(g):
        c0 = g * NBUF
        gather(c0, 0).wait()
        scatter(c0, 0).start()

        @pl.when(g > 0)
        def _():
            scatter(c0 - 1, 1).wait()

        gather(c0 + 1, 1).start()
        gather(c0 + 1, 1).wait()
        scatter(c0 + 1, 1).start()
        scatter(c0, 0).wait()

        @pl.when(g < NGROUP - 1)
        def _():
            gather(c0 + 2, 0).start()

    scatter(NCHUNK - 1, 1).wait()


@jax.jit
def _sc_gather(tmp_enc, tidx):
    mesh = plsc.VectorSubcoreMesh(core_axis_name="c", subcore_axis_name="s")
    return pl.kernel(
        _gather_body,
        out_type=jax.ShapeDtypeStruct((B, D), jnp.float32),
        mesh=mesh,
        scratch_types=[
            pltpu.VMEM_SHARED((1024, D), jnp.float32),
            pltpu.VMEM((BPW,), jnp.int32),
            pltpu.VMEM((CHUNK, D), jnp.float32),
            pltpu.VMEM((CHUNK, D), jnp.float32),
            pltpu.SemaphoreType.DMA,
            pltpu.SemaphoreType.DMA,
            pltpu.SemaphoreType.DMA,
        ],
    )(tmp_enc, tidx)


def kernel(time, tmp_enc):
    tidx = time.T.reshape(-1).astype(jnp.int32)
    out = _sc_gather(tmp_enc, tidx)
    return out.reshape(T, NB, D).transpose(1, 0, 2)
